# Initial kernel scaffold; baseline (speedup 1.0000x reference)
import jax
import jax.numpy as jnp
from jax.experimental import pallas as pl


def kernel(inputs, pos, Wq, bq, Wk, bk, Wv, bv, pe_W1, pe_b1, pe_W2, pe_b2, gru_Wih, gru_Whh, gru_bih, gru_bhh, mlp_W1, mlp_b1, mlp_W2, mlp_b2, ln_in_g, ln_in_b, ln_s_g, ln_s_b, ln_m_g, ln_m_b):
    raise NotImplementedError("write your pallas kernel here")


# speedup vs baseline: 1.4153x; 1.4153x over previous
import jax
import jax.numpy as jnp
from jax.experimental import pallas as pl

_B, _N, _D = 4, 4096, 256
_S, _K, _ITERS, _H = 64, 16, 3, 128
_PREC = jax.lax.Precision.HIGHEST


def _ln(x, g, b, eps=1e-5):
    m = jnp.mean(x, axis=-1, keepdims=True)
    xc = x - m
    v = jnp.mean(xc * xc, axis=-1, keepdims=True)
    return xc / jnp.sqrt(v + eps) * g + b


def _dot_nt(a, b):
    return jax.lax.dot_general(a, b, (((1,), (1,)), ((), ())), precision=_PREC)


def _dot_nn(a, b):
    return jax.lax.dot_general(a, b, (((1,), (0,)), ((), ())), precision=_PREC)


def _fused_body(inputs_ref, posT_ref,
                Wq_ref, bq_ref, Wk_ref, bk_ref, Wv_ref, bv_ref,
                pe_W1T_ref, pe_b1_ref, pe_W2_ref, pe_b2_ref,
                gru_Wih_ref, gru_Whh_ref, gru_bih_ref, gru_bhh_ref,
                mlp_W1_ref, mlp_b1_ref, mlp_W2_ref, mlp_b2_ref,
                ln_in_g_ref, ln_in_b_ref, ln_s_g_ref, ln_s_b_ref,
                ln_m_g_ref, ln_m_b_ref,
                slots_out_ref, spos_out_ref):
    x = inputs_ref[0]
    posT = posT_ref[0]
    px = posT[0:1, :]
    py = posT[1:2, :]
    pz = posT[2:3, :]

    xn = _ln(x, ln_in_g_ref[...], ln_in_b_ref[...])

    kfeat = _dot_nt(xn, Wk_ref[...]) + bk_ref[...]
    vfeat = _dot_nt(xn, Wv_ref[...]) + bv_ref[...]
    ksum = _dot_nt(jnp.ones((1, _D), jnp.float32), kfeat)

    col1 = jax.lax.broadcasted_iota(jnp.int32, (1, _N), 1)
    colS = jax.lax.broadcasted_iota(jnp.int32, (_S, _N), 1)
    rowS = jax.lax.broadcasted_iota(jnp.int32, (_S, _N), 0)

    def fps_body(t, carry):
        dist, far, oh = carry
        sel = col1 == far
        oh = oh + jnp.where((rowS == t) & (colS == far), 1.0, 0.0)
        cx = jnp.sum(jnp.where(sel, px, 0.0))
        cy = jnp.sum(jnp.where(sel, py, 0.0))
        cz = jnp.sum(jnp.where(sel, pz, 0.0))
        d = (px - cx) ** 2 + (py - cy) ** 2 + (pz - cz) ** 2
        dist = jnp.minimum(dist, d)
        m = jnp.max(dist)
        far = jnp.min(jnp.where(dist == m, col1, _N)).astype(jnp.int32)
        return dist, far, oh

    dist0 = jnp.full((1, _N), 1e10, jnp.float32)
    oh0 = jnp.zeros((_S, _N), jnp.float32)
    _, _, OH = jax.lax.fori_loop(0, _S, fps_body, (dist0, jnp.int32(0), oh0))

    slots = _dot_nn(OH, xn)
    spos = _dot_nt(OH, posT)
    spx = spos[:, 0:1]
    spy = spos[:, 1:2]
    spz = spos[:, 2:3]

    pospk = jnp.concatenate([posT, ksum], axis=0)

    w1x = pe_W1T_ref[0:1, :]
    w1y = pe_W1T_ref[1:2, :]
    w1z = pe_W1T_ref[2:3, :]
    pe_c = jnp.sum(pe_W2_ref[...], axis=0, keepdims=True)
    pe_const = jnp.sum(pe_b2_ref[...])

    for _ in range(_ITERS):
        slots_prev = slots
        sn = _ln(slots, ln_s_g_ref[...], ln_s_b_ref[...])
        q = _dot_nt(sn, Wq_ref[...]) + bq_ref[...]
        qsum = jnp.sum(q, axis=1, keepdims=True)

        dmat = (spx - px) ** 2 + (spy - py) ** 2 + (spz - pz) ** 2

        work = dmat
        sels = []
        pes_cols = []
        ksn_cols = []
        for _r in range(_K):
            mn = jnp.min(work, axis=1, keepdims=True)
            sel = jnp.min(jnp.where(work == mn, colS, _N), axis=1, keepdims=True)
            ohr = (colS == sel).astype(jnp.float32)
            g = _dot_nt(ohr, pospk)
            work = jnp.where(colS == sel, 1e30, work)
            sels.append(sel)
            ksn_cols.append(g[:, 3:4])
            hr = jax.nn.relu((spx - g[:, 0:1]) * w1x + (spy - g[:, 1:2]) * w1y
                             + (spz - g[:, 2:3]) * w1z + pe_b1_ref[...])
            pes_cols.append(jnp.sum(hr * pe_c, axis=1, keepdims=True))

        ksn = jnp.concatenate(ksn_cols, axis=1)
        pesum = jnp.concatenate(pes_cols, axis=1) + pe_const

        scores = qsum - ksn + pesum
        smax = jnp.max(scores, axis=1, keepdims=True)
        e = jnp.exp(scores - smax)
        a = e / jnp.sum(e, axis=1, keepdims=True)
        a = a / (jnp.sum(a, axis=0, keepdims=True) + 1e-6)

        amat = jnp.zeros((_S, _N), jnp.float32)
        for r in range(_K):
            amat = amat + jnp.where(colS == sels[r], a[:, r:r + 1], 0.0)
        upd = _dot_nn(amat, vfeat)

        gi = _dot_nt(upd, gru_Wih_ref[...]) + gru_bih_ref[...]
        gh = _dot_nt(slots_prev, gru_Whh_ref[...]) + gru_bhh_ref[...]
        i_r = gi[:, :_D]
        i_z = gi[:, _D:2 * _D]
        i_n = gi[:, 2 * _D:]
        h_r = gh[:, :_D]
        h_z = gh[:, _D:2 * _D]
        h_n = gh[:, 2 * _D:]
        r_g = jax.nn.sigmoid(i_r + h_r)
        z_g = jax.nn.sigmoid(i_z + h_z)
        n_g = jnp.tanh(i_n + r_g * h_n)
        slots = (1.0 - z_g) * n_g + z_g * slots_prev

        mid = jax.nn.relu(
            _dot_nt(_ln(slots, ln_m_g_ref[...], ln_m_b_ref[...]), mlp_W1_ref[...])
            + mlp_b1_ref[...])
        slots = slots + _dot_nt(mid, mlp_W2_ref[...]) + mlp_b2_ref[...]

    slots_out_ref[0] = slots
    spos_out_ref[0] = spos


def kernel(inputs, pos, Wq, bq, Wk, bk, Wv, bv, pe_W1, pe_b1, pe_W2, pe_b2,
           gru_Wih, gru_Whh, gru_bih, gru_bhh, mlp_W1, mlp_b1, mlp_W2, mlp_b2,
           ln_in_g, ln_in_b, ln_s_g, ln_s_b, ln_m_g, ln_m_b):
    posT = jnp.transpose(pos, (0, 2, 1))
    pe_W1T = pe_W1.T
    row = lambda v: v.reshape(1, -1)

    full = lambda shape: pl.BlockSpec(shape, lambda b: (0,) * len(shape))
    in_specs = [
        pl.BlockSpec((1, _N, _D), lambda b: (b, 0, 0)),
        pl.BlockSpec((1, 3, _N), lambda b: (b, 0, 0)),
        full((_D, _D)), full((1, _D)),
        full((_D, _D)), full((1, _D)),
        full((_D, _D)), full((1, _D)),
        full((3, _D)), full((1, _D)),
        full((_D, _D)), full((1, _D)),
        full((3 * _D, _D)), full((3 * _D, _D)),
        full((1, 3 * _D)), full((1, 3 * _D)),
        full((_H, _D)), full((1, _H)),
        full((_D, _H)), full((1, _D)),
        full((1, _D)), full((1, _D)),
        full((1, _D)), full((1, _D)),
        full((1, _D)), full((1, _D)),
    ]
    out_specs = [
        pl.BlockSpec((1, _S, _D), lambda b: (b, 0, 0)),
        pl.BlockSpec((1, _S, 3), lambda b: (b, 0, 0)),
    ]
    slots, spos = pl.pallas_call(
        _fused_body,
        grid=(_B,),
        in_specs=in_specs,
        out_specs=out_specs,
        out_shape=[
            jax.ShapeDtypeStruct((_B, _S, _D), jnp.float32),
            jax.ShapeDtypeStruct((_B, _S, 3), jnp.float32),
        ],
    )(inputs, posT,
      Wq, row(bq), Wk, row(bk), Wv, row(bv),
      pe_W1T, row(pe_b1), pe_W2, row(pe_b2),
      gru_Wih, gru_Whh, row(gru_bih), row(gru_bhh),
      mlp_W1, row(mlp_b1), mlp_W2, row(mlp_b2),
      row(ln_in_g), row(ln_in_b), row(ln_s_g), row(ln_s_b),
      row(ln_m_g), row(ln_m_b))
    return slots, spos


# speedup vs baseline: 2.2487x; 1.5889x over previous
import jax
import jax.numpy as jnp
from jax.experimental import pallas as pl

_B, _N, _D = 4, 4096, 256
_S, _K, _ITERS, _H = 64, 16, 3, 128
_BN = _B * _N
_BS = _B * _S
_CH = 2048
_NC = _BN // _CH
_PREC = jax.lax.Precision.HIGHEST


def _ln(x, g, b, eps=1e-5):
    m = jnp.mean(x, axis=-1, keepdims=True)
    xc = x - m
    v = jnp.mean(xc * xc, axis=-1, keepdims=True)
    return xc / jnp.sqrt(v + eps) * g + b


def _dot_nt(a, b):
    return jax.lax.dot_general(a, b, (((1,), (1,)), ((), ())), precision=_PREC)


def _dot_nn(a, b):
    return jax.lax.dot_general(a, b, (((1,), (0,)), ((), ())), precision=_PREC)


def _proj_body(x_ref, Wv_ref, bv_ref, Wk_ref, bk_ref, g_ref, b_ref,
               v_ref, k_ref):
    xn = _ln(x_ref[...], g_ref[...], b_ref[...])
    v_ref[...] = _dot_nt(xn, Wv_ref[...]) + bv_ref[...]
    wkc = jnp.sum(Wk_ref[...], axis=0, keepdims=True)
    k_ref[0] = _dot_nt(wkc, xn) + jnp.sum(bk_ref[...])


def _k1(inp2, Wv, bv, Wk, bk, g, b):
    return pl.pallas_call(
        _proj_body,
        grid=(_NC,),
        in_specs=[
            pl.BlockSpec((_CH, _D), lambda c: (c, 0)),
            pl.BlockSpec((_D, _D), lambda c: (0, 0)),
            pl.BlockSpec((1, _D), lambda c: (0, 0)),
            pl.BlockSpec((_D, _D), lambda c: (0, 0)),
            pl.BlockSpec((1, _D), lambda c: (0, 0)),
            pl.BlockSpec((1, _D), lambda c: (0, 0)),
            pl.BlockSpec((1, _D), lambda c: (0, 0)),
        ],
        out_specs=[
            pl.BlockSpec((_CH, _D), lambda c: (c, 0)),
            pl.BlockSpec((1, 1, _CH), lambda c: (c, 0, 0)),
        ],
        out_shape=[
            jax.ShapeDtypeStruct((_BN, _D), jnp.float32),
            jax.ShapeDtypeStruct((_NC, 1, _CH), jnp.float32),
        ],
    )(inp2, Wv, bv, Wk, bk, g, b)


def _fps_body(pos3_ref, oh_ref, spos_ref):
    colN = jax.lax.broadcasted_iota(jnp.int32, (_B, _N), 1)
    colS = jax.lax.broadcasted_iota(jnp.int32, (_S, _N), 1)
    rowS = jax.lax.broadcasted_iota(jnp.int32, (_S, _N), 0)
    px = jnp.concatenate([pos3_ref[b, 0:1, :] for b in range(_B)], axis=0)
    py = jnp.concatenate([pos3_ref[b, 1:2, :] for b in range(_B)], axis=0)
    pz = jnp.concatenate([pos3_ref[b, 2:3, :] for b in range(_B)], axis=0)

    oh_ref[...] = jnp.zeros((_S, _BN), jnp.float32)

    def fps_body(t, carry):
        dist, far = carry
        rowt = rowS == t
        for b in range(_B):
            oh_ref[:, b * _N:(b + 1) * _N] += jnp.where(
                rowt & (colS == far[b:b + 1, 0:1]), 1.0, 0.0)
        selN = colN == far
        cx = jnp.sum(jnp.where(selN, px, 0.0), axis=1, keepdims=True)
        cy = jnp.sum(jnp.where(selN, py, 0.0), axis=1, keepdims=True)
        cz = jnp.sum(jnp.where(selN, pz, 0.0), axis=1, keepdims=True)
        d = (px - cx) ** 2 + (py - cy) ** 2 + (pz - cz) ** 2
        dist = jnp.minimum(dist, d)
        m = jnp.max(dist, axis=1, keepdims=True)
        far = jnp.min(jnp.where(dist == m, colN, _N), axis=1, keepdims=True)
        return dist, far.astype(jnp.int32)

    dist0 = jnp.full((_B, _N), 1e10, jnp.float32)
    far0 = jnp.zeros((_B, 1), jnp.int32)
    jax.lax.fori_loop(0, _S, fps_body, (dist0, far0))

    for b in range(_B):
        spos_ref[b * _S:(b + 1) * _S, :] = _dot_nt(
            oh_ref[:, b * _N:(b + 1) * _N], pos3_ref[b])


def _k2(pos3):
    return pl.pallas_call(
        _fps_body,
        out_shape=[
            jax.ShapeDtypeStruct((_S, _BN), jnp.float32),
            jax.ShapeDtypeStruct((_BS, 3), jnp.float32),
        ],
    )(pos3)


def _slot_body(oh_ref, x_ref, g_ref, b_ref, s_ref):
    raw = _dot_nn(oh_ref[...], x_ref[...])
    s_ref[...] = _ln(raw, g_ref[...], b_ref[...])


def _k3(oh, inp2, g, b):
    return pl.pallas_call(
        _slot_body,
        grid=(_B,),
        in_specs=[
            pl.BlockSpec((_S, _N), lambda i: (0, i)),
            pl.BlockSpec((_N, _D), lambda i: (i, 0)),
            pl.BlockSpec((1, _D), lambda i: (0, 0)),
            pl.BlockSpec((1, _D), lambda i: (0, 0)),
        ],
        out_specs=pl.BlockSpec((_S, _D), lambda i: (i, 0)),
        out_shape=jax.ShapeDtypeStruct((_BS, _D), jnp.float32),
    )(oh, inp2, g, b)


def _topk_body(pos3_ref, spos_ref, ksum_ref, pe_W1T_ref, pe_b1_ref,
               pe_W2_ref, pe_b2_ref, sel_ref, ksn_ref, pes_ref):
    px = pos3_ref[0, 0:1, :]
    py = pos3_ref[0, 1:2, :]
    pz = pos3_ref[0, 2:3, :]
    spx = spos_ref[:, 0:1]
    spy = spos_ref[:, 1:2]
    spz = spos_ref[:, 2:3]
    colS = jax.lax.broadcasted_iota(jnp.int32, (_S, _N), 1)

    work = (spx - px) ** 2 + (spy - py) ** 2 + (spz - pz) ** 2
    tab = jnp.concatenate([px, py, pz, ksum_ref[0]], axis=0)

    w1x = pe_W1T_ref[0:1, :]
    w1y = pe_W1T_ref[1:2, :]
    w1z = pe_W1T_ref[2:3, :]
    pe_b1 = pe_b1_ref[...]
    pe_c = jnp.sum(pe_W2_ref[...], axis=0, keepdims=True)
    pe_const = jnp.sum(pe_b2_ref[...])

    for r in range(_K):
        mn = jnp.min(work, axis=1, keepdims=True)
        sel = jnp.min(jnp.where(work == mn, colS, _N), axis=1, keepdims=True)
        ohr = (colS == sel).astype(jnp.float32)
        gf = _dot_nt(ohr, tab)
        work = jnp.where(colS == sel, 1e30, work)
        sel_ref[:, r:r + 1] = sel
        ksn_ref[:, r:r + 1] = gf[:, 3:4]
        hr = jax.nn.relu((spx - gf[:, 0:1]) * w1x + (spy - gf[:, 1:2]) * w1y
                         + (spz - gf[:, 2:3]) * w1z + pe_b1)
        pes_ref[:, r:r + 1] = jnp.sum(hr * pe_c, axis=1, keepdims=True) + pe_const


def _k4(pos3, spos, ksum3, pe_W1T, pe_b1, pe_W2, pe_b2):
    return pl.pallas_call(
        _topk_body,
        grid=(_B,),
        in_specs=[
            pl.BlockSpec((1, 3, _N), lambda i: (i, 0, 0)),
            pl.BlockSpec((_S, 3), lambda i: (i, 0)),
            pl.BlockSpec((1, 1, _N), lambda i: (i, 0, 0)),
            pl.BlockSpec((3, _D), lambda i: (0, 0)),
            pl.BlockSpec((1, _D), lambda i: (0, 0)),
            pl.BlockSpec((_D, _D), lambda i: (0, 0)),
            pl.BlockSpec((1, _D), lambda i: (0, 0)),
        ],
        out_specs=[
            pl.BlockSpec((_S, _K), lambda i: (i, 0)),
            pl.BlockSpec((_S, _K), lambda i: (i, 0)),
            pl.BlockSpec((_S, _K), lambda i: (i, 0)),
        ],
        out_shape=[
            jax.ShapeDtypeStruct((_BS, _K), jnp.int32),
            jax.ShapeDtypeStruct((_BS, _K), jnp.float32),
            jax.ShapeDtypeStruct((_BS, _K), jnp.float32),
        ],
    )(pos3, spos, ksum3, pe_W1T, pe_b1, pe_W2, pe_b2)


def _iter_body(slots0_ref, sel_ref, ksn_ref, pes_ref, vfeat_ref,
               Wq_ref, bq_ref,
               gru_Wih_ref, gru_Whh_ref, gru_bih_ref, gru_bhh_ref,
               mlp_W1_ref, mlp_b1_ref, mlp_W2_ref, mlp_b2_ref,
               ln_s_g_ref, ln_s_b_ref, ln_m_g_ref, ln_m_b_ref,
               out_ref):
    slots = slots0_ref[...]
    ksn = ksn_ref[...]
    pesum = pes_ref[...]
    selb = sel_ref[...]
    colS = jax.lax.broadcasted_iota(jnp.int32, (_S, _N), 1)

    for _ in range(_ITERS):
        slots_prev = slots
        sn = _ln(slots, ln_s_g_ref[...], ln_s_b_ref[...])
        q = _dot_nt(sn, Wq_ref[...]) + bq_ref[...]
        qsum = jnp.sum(q, axis=1, keepdims=True)

        scores = qsum - ksn + pesum
        smax = jnp.max(scores, axis=1, keepdims=True)
        e = jnp.exp(scores - smax)
        a = e / jnp.sum(e, axis=1, keepdims=True)
        a = a / (jnp.sum(a, axis=0, keepdims=True) + 1e-6)

        amat = jnp.zeros((_S, _N), jnp.float32)
        for r in range(_K):
            amat = amat + jnp.where(colS == selb[:, r:r + 1],
                                    a[:, r:r + 1], 0.0)
        upd = _dot_nn(amat, vfeat_ref[...])

        gi = _dot_nt(upd, gru_Wih_ref[...]) + gru_bih_ref[...]
        gh = _dot_nt(slots_prev, gru_Whh_ref[...]) + gru_bhh_ref[...]
        i_r = gi[:, :_D]
        i_z = gi[:, _D:2 * _D]
        i_n = gi[:, 2 * _D:]
        h_r = gh[:, :_D]
        h_z = gh[:, _D:2 * _D]
        h_n = gh[:, 2 * _D:]
        r_g = jax.nn.sigmoid(i_r + h_r)
        z_g = jax.nn.sigmoid(i_z + h_z)
        n_g = jnp.tanh(i_n + r_g * h_n)
        slots = (1.0 - z_g) * n_g + z_g * slots_prev

        mid = jax.nn.relu(
            _dot_nt(_ln(slots, ln_m_g_ref[...], ln_m_b_ref[...]), mlp_W1_ref[...])
            + mlp_b1_ref[...])
        slots = slots + _dot_nt(mid, mlp_W2_ref[...]) + mlp_b2_ref[...]

    out_ref[...] = slots


def _k5(slots0, sel, ksn, pes, vfeat, Wq, bq, gru_Wih, gru_Whh, gru_bih,
        gru_bhh, mlp_W1, mlp_b1, mlp_W2, mlp_b2, ln_s_g, ln_s_b,
        ln_m_g, ln_m_b):
    w = lambda shape: pl.BlockSpec(shape, lambda i: (0,) * len(shape))
    return pl.pallas_call(
        _iter_body,
        grid=(_B,),
        in_specs=[
            pl.BlockSpec((_S, _D), lambda i: (i, 0)),
            pl.BlockSpec((_S, _K), lambda i: (i, 0)),
            pl.BlockSpec((_S, _K), lambda i: (i, 0)),
            pl.BlockSpec((_S, _K), lambda i: (i, 0)),
            pl.BlockSpec((_N, _D), lambda i: (i, 0)),
            w((_D, _D)), w((1, _D)),
            w((3 * _D, _D)), w((3 * _D, _D)), w((1, 3 * _D)), w((1, 3 * _D)),
            w((_H, _D)), w((1, _H)), w((_D, _H)), w((1, _D)),
            w((1, _D)), w((1, _D)), w((1, _D)), w((1, _D)),
        ],
        out_specs=pl.BlockSpec((_S, _D), lambda i: (i, 0)),
        out_shape=jax.ShapeDtypeStruct((_BS, _D), jnp.float32),
    )(slots0, sel, ksn, pes, vfeat, Wq, bq, gru_Wih, gru_Whh, gru_bih,
      gru_bhh, mlp_W1, mlp_b1, mlp_W2, mlp_b2, ln_s_g, ln_s_b,
      ln_m_g, ln_m_b)


def kernel(inputs, pos, Wq, bq, Wk, bk, Wv, bv, pe_W1, pe_b1, pe_W2, pe_b2,
           gru_Wih, gru_Whh, gru_bih, gru_bhh, mlp_W1, mlp_b1, mlp_W2, mlp_b2,
           ln_in_g, ln_in_b, ln_s_g, ln_s_b, ln_m_g, ln_m_b):
    inp2 = inputs.reshape(_BN, _D)
    pos3 = jnp.transpose(pos, (0, 2, 1))
    row = lambda v: v.reshape(1, -1)

    vfeat, kt = _k1(inp2, Wv, row(bv), Wk, row(bk), row(ln_in_g), row(ln_in_b))
    ksum3 = kt.reshape(_B, 1, _N)
    oh, spos = _k2(pos3)
    slots0 = _k3(oh, inp2, row(ln_in_g), row(ln_in_b))
    sel, ksn, pes = _k4(pos3, spos, ksum3, pe_W1.T, row(pe_b1), pe_W2,
                        row(pe_b2))
    slots = _k5(slots0, sel, ksn, pes, vfeat, Wq, row(bq),
                gru_Wih, gru_Whh, row(gru_bih), row(gru_bhh),
                mlp_W1, row(mlp_b1), mlp_W2, row(mlp_b2),
                row(ln_s_g), row(ln_s_b), row(ln_m_g), row(ln_m_b))
    return slots.reshape(_B, _S, _D), spos.reshape(_B, _S, 3)


# speedup vs baseline: 2.6597x; 1.1828x over previous
import jax
import jax.numpy as jnp
from jax.experimental import pallas as pl

_B, _N, _D = 4, 4096, 256
_S, _K, _ITERS, _H = 64, 16, 3, 128
_BN = _B * _N
_BS = _B * _S
_CH = 2048
_NC = _BN // _CH
_PREC = jax.lax.Precision.HIGHEST


def _ln(x, g, b, eps=1e-5):
    m = jnp.mean(x, axis=-1, keepdims=True)
    xc = x - m
    v = jnp.mean(xc * xc, axis=-1, keepdims=True)
    return xc / jnp.sqrt(v + eps) * g + b


def _dot_nt(a, b):
    return jax.lax.dot_general(a, b, (((1,), (1,)), ((), ())), precision=_PREC)


def _dot_nn(a, b):
    return jax.lax.dot_general(a, b, (((1,), (0,)), ((), ())), precision=_PREC)


def _proj_body(x_ref, Wv_ref, bv_ref, Wk_ref, bk_ref, g_ref, b_ref,
               v_ref, k_ref):
    xn = _ln(x_ref[...], g_ref[...], b_ref[...])
    v_ref[...] = _dot_nt(xn, Wv_ref[...]) + bv_ref[...]
    wkc = jnp.sum(Wk_ref[...], axis=0, keepdims=True)
    k_ref[0] = _dot_nt(wkc, xn) + jnp.sum(bk_ref[...])


def _k1(inp2, Wv, bv, Wk, bk, g, b):
    return pl.pallas_call(
        _proj_body,
        grid=(_NC,),
        in_specs=[
            pl.BlockSpec((_CH, _D), lambda c: (c, 0)),
            pl.BlockSpec((_D, _D), lambda c: (0, 0)),
            pl.BlockSpec((1, _D), lambda c: (0, 0)),
            pl.BlockSpec((_D, _D), lambda c: (0, 0)),
            pl.BlockSpec((1, _D), lambda c: (0, 0)),
            pl.BlockSpec((1, _D), lambda c: (0, 0)),
            pl.BlockSpec((1, _D), lambda c: (0, 0)),
        ],
        out_specs=[
            pl.BlockSpec((_CH, _D), lambda c: (c, 0)),
            pl.BlockSpec((1, 1, _CH), lambda c: (c, 0, 0)),
        ],
        out_shape=[
            jax.ShapeDtypeStruct((_BN, _D), jnp.float32),
            jax.ShapeDtypeStruct((_NC, 1, _CH), jnp.float32),
        ],
    )(inp2, Wv, bv, Wk, bk, g, b)


def _fps_body(pos3_ref, fars_ref, spos_ref):
    colN = jax.lax.broadcasted_iota(jnp.int32, (_B, _N), 1)
    colS = jax.lax.broadcasted_iota(jnp.int32, (_S, _N), 1)
    colF = jax.lax.broadcasted_iota(jnp.int32, (_S, 128), 1)
    rowF = jax.lax.broadcasted_iota(jnp.int32, (_S, 128), 0)
    px = jnp.concatenate([pos3_ref[b, 0:1, :] for b in range(_B)], axis=0)
    py = jnp.concatenate([pos3_ref[b, 1:2, :] for b in range(_B)], axis=0)
    pz = jnp.concatenate([pos3_ref[b, 2:3, :] for b in range(_B)], axis=0)

    fars_ref[...] = jnp.zeros((_S, 128), jnp.float32)

    def fps_body(t, carry):
        dist, far = carry
        rowt = rowF == t
        upd = jnp.zeros((_S, 128), jnp.float32)
        for b in range(_B):
            fb = far[b:b + 1, 0:1].astype(jnp.float32)
            upd = upd + jnp.where(rowt & (colF == b), fb, 0.0)
        fars_ref[...] += upd
        selN = colN == far
        cx = jnp.sum(jnp.where(selN, px, 0.0), axis=1, keepdims=True)
        cy = jnp.sum(jnp.where(selN, py, 0.0), axis=1, keepdims=True)
        cz = jnp.sum(jnp.where(selN, pz, 0.0), axis=1, keepdims=True)
        d = (px - cx) ** 2 + (py - cy) ** 2 + (pz - cz) ** 2
        dist = jnp.minimum(dist, d)
        m = jnp.max(dist, axis=1, keepdims=True)
        far = jnp.min(jnp.where(dist == m, colN, _N), axis=1, keepdims=True)
        return dist, far.astype(jnp.int32)

    dist0 = jnp.full((_B, _N), 1e10, jnp.float32)
    far0 = jnp.zeros((_B, 1), jnp.int32)
    jax.lax.fori_loop(0, _S, fps_body, (dist0, far0))

    for b in range(_B):
        idx = fars_ref[:, b:b + 1].astype(jnp.int32)
        ohb = (colS == idx).astype(jnp.float32)
        spos_ref[b * _S:(b + 1) * _S, :] = _dot_nt(ohb, pos3_ref[b])


def _k2(pos3):
    return pl.pallas_call(
        _fps_body,
        out_shape=[
            jax.ShapeDtypeStruct((_S, 128), jnp.float32),
            jax.ShapeDtypeStruct((_BS, 3), jnp.float32),
        ],
    )(pos3)


def _slot_body(fars_ref, x_ref, g_ref, b_ref, s_ref):
    pid = pl.program_id(0)
    idx = jnp.zeros((_S, 1), jnp.float32)
    for b in range(_B):
        idx = idx + jnp.where(pid == b, fars_ref[:, b:b + 1], 0.0)
    colS = jax.lax.broadcasted_iota(jnp.int32, (_S, _N), 1)
    oh = (colS == idx.astype(jnp.int32)).astype(jnp.float32)
    raw = _dot_nn(oh, x_ref[...])
    s_ref[...] = _ln(raw, g_ref[...], b_ref[...])


def _k3(fars, inp2, g, b):
    return pl.pallas_call(
        _slot_body,
        grid=(_B,),
        in_specs=[
            pl.BlockSpec((_S, 128), lambda i: (0, 0)),
            pl.BlockSpec((_N, _D), lambda i: (i, 0)),
            pl.BlockSpec((1, _D), lambda i: (0, 0)),
            pl.BlockSpec((1, _D), lambda i: (0, 0)),
        ],
        out_specs=pl.BlockSpec((_S, _D), lambda i: (i, 0)),
        out_shape=jax.ShapeDtypeStruct((_BS, _D), jnp.float32),
    )(fars, inp2, g, b)


def _topk_body(pos3_ref, spos_ref, ksum_ref, pe_W1T_ref, pe_b1_ref,
               pe_W2_ref, pe_b2_ref, sel_ref, ksn_ref, pes_ref):
    px = pos3_ref[0, 0:1, :]
    py = pos3_ref[0, 1:2, :]
    pz = pos3_ref[0, 2:3, :]
    spx = spos_ref[:, 0:1]
    spy = spos_ref[:, 1:2]
    spz = spos_ref[:, 2:3]
    colS = jax.lax.broadcasted_iota(jnp.int32, (_S, _N), 1)

    work = (spx - px) ** 2 + (spy - py) ** 2 + (spz - pz) ** 2
    tab = jnp.concatenate([px, py, pz, ksum_ref[0]], axis=0)

    w1x = pe_W1T_ref[0:1, :]
    w1y = pe_W1T_ref[1:2, :]
    w1z = pe_W1T_ref[2:3, :]
    pe_b1 = pe_b1_ref[...]
    pe_c = jnp.sum(pe_W2_ref[...], axis=0, keepdims=True)
    pe_const = jnp.sum(pe_b2_ref[...])

    for r in range(_K):
        mn = jnp.min(work, axis=1, keepdims=True)
        sel = jnp.min(jnp.where(work == mn, colS, _N), axis=1, keepdims=True)
        ohr = (colS == sel).astype(jnp.float32)
        gf = _dot_nt(ohr, tab)
        work = jnp.where(colS == sel, 1e30, work)
        sel_ref[:, r:r + 1] = sel
        ksn_ref[:, r:r + 1] = gf[:, 3:4]
        hr = jax.nn.relu((spx - gf[:, 0:1]) * w1x + (spy - gf[:, 1:2]) * w1y
                         + (spz - gf[:, 2:3]) * w1z + pe_b1)
        pes_ref[:, r:r + 1] = jnp.sum(hr * pe_c, axis=1, keepdims=True) + pe_const


def _k4(pos3, spos, ksum3, pe_W1T, pe_b1, pe_W2, pe_b2):
    return pl.pallas_call(
        _topk_body,
        grid=(_B,),
        in_specs=[
            pl.BlockSpec((1, 3, _N), lambda i: (i, 0, 0)),
            pl.BlockSpec((_S, 3), lambda i: (i, 0)),
            pl.BlockSpec((1, 1, _N), lambda i: (i, 0, 0)),
            pl.BlockSpec((3, _D), lambda i: (0, 0)),
            pl.BlockSpec((1, _D), lambda i: (0, 0)),
            pl.BlockSpec((_D, _D), lambda i: (0, 0)),
            pl.BlockSpec((1, _D), lambda i: (0, 0)),
        ],
        out_specs=[
            pl.BlockSpec((_S, _K), lambda i: (i, 0)),
            pl.BlockSpec((_S, _K), lambda i: (i, 0)),
            pl.BlockSpec((_S, _K), lambda i: (i, 0)),
        ],
        out_shape=[
            jax.ShapeDtypeStruct((_BS, _K), jnp.int32),
            jax.ShapeDtypeStruct((_BS, _K), jnp.float32),
            jax.ShapeDtypeStruct((_BS, _K), jnp.float32),
        ],
    )(pos3, spos, ksum3, pe_W1T, pe_b1, pe_W2, pe_b2)


def _iter_body(slots0_ref, sel_ref, ksn_ref, pes_ref, vfeat_ref,
               Wq_ref, bq_ref,
               gru_Wih_ref, gru_Whh_ref, gru_bih_ref, gru_bhh_ref,
               mlp_W1_ref, mlp_b1_ref, mlp_W2_ref, mlp_b2_ref,
               ln_s_g_ref, ln_s_b_ref, ln_m_g_ref, ln_m_b_ref,
               out_ref):
    slots = slots0_ref[...]
    ksn = ksn_ref[...]
    pesum = pes_ref[...]
    selb = sel_ref[...]
    colS = jax.lax.broadcasted_iota(jnp.int32, (_S, _N), 1)

    for _ in range(_ITERS):
        slots_prev = slots
        sn = _ln(slots, ln_s_g_ref[...], ln_s_b_ref[...])
        q = _dot_nt(sn, Wq_ref[...]) + bq_ref[...]
        qsum = jnp.sum(q, axis=1, keepdims=True)

        scores = qsum - ksn + pesum
        smax = jnp.max(scores, axis=1, keepdims=True)
        e = jnp.exp(scores - smax)
        a = e / jnp.sum(e, axis=1, keepdims=True)
        a = a / (jnp.sum(a, axis=0, keepdims=True) + 1e-6)

        amat = jnp.zeros((_S, _N), jnp.float32)
        for r in range(_K):
            amat = amat + jnp.where(colS == selb[:, r:r + 1],
                                    a[:, r:r + 1], 0.0)
        upd = _dot_nn(amat, vfeat_ref[...])

        gi = _dot_nt(upd, gru_Wih_ref[...]) + gru_bih_ref[...]
        gh = _dot_nt(slots_prev, gru_Whh_ref[...]) + gru_bhh_ref[...]
        i_r = gi[:, :_D]
        i_z = gi[:, _D:2 * _D]
        i_n = gi[:, 2 * _D:]
        h_r = gh[:, :_D]
        h_z = gh[:, _D:2 * _D]
        h_n = gh[:, 2 * _D:]
        r_g = jax.nn.sigmoid(i_r + h_r)
        z_g = jax.nn.sigmoid(i_z + h_z)
        n_g = jnp.tanh(i_n + r_g * h_n)
        slots = (1.0 - z_g) * n_g + z_g * slots_prev

        mid = jax.nn.relu(
            _dot_nt(_ln(slots, ln_m_g_ref[...], ln_m_b_ref[...]), mlp_W1_ref[...])
            + mlp_b1_ref[...])
        slots = slots + _dot_nt(mid, mlp_W2_ref[...]) + mlp_b2_ref[...]

    out_ref[...] = slots


def _k5(slots0, sel, ksn, pes, vfeat, Wq, bq, gru_Wih, gru_Whh, gru_bih,
        gru_bhh, mlp_W1, mlp_b1, mlp_W2, mlp_b2, ln_s_g, ln_s_b,
        ln_m_g, ln_m_b):
    w = lambda shape: pl.BlockSpec(shape, lambda i: (0,) * len(shape))
    return pl.pallas_call(
        _iter_body,
        grid=(_B,),
        in_specs=[
            pl.BlockSpec((_S, _D), lambda i: (i, 0)),
            pl.BlockSpec((_S, _K), lambda i: (i, 0)),
            pl.BlockSpec((_S, _K), lambda i: (i, 0)),
            pl.BlockSpec((_S, _K), lambda i: (i, 0)),
            pl.BlockSpec((_N, _D), lambda i: (i, 0)),
            w((_D, _D)), w((1, _D)),
            w((3 * _D, _D)), w((3 * _D, _D)), w((1, 3 * _D)), w((1, 3 * _D)),
            w((_H, _D)), w((1, _H)), w((_D, _H)), w((1, _D)),
            w((1, _D)), w((1, _D)), w((1, _D)), w((1, _D)),
        ],
        out_specs=pl.BlockSpec((_S, _D), lambda i: (i, 0)),
        out_shape=jax.ShapeDtypeStruct((_BS, _D), jnp.float32),
    )(slots0, sel, ksn, pes, vfeat, Wq, bq, gru_Wih, gru_Whh, gru_bih,
      gru_bhh, mlp_W1, mlp_b1, mlp_W2, mlp_b2, ln_s_g, ln_s_b,
      ln_m_g, ln_m_b)


def kernel(inputs, pos, Wq, bq, Wk, bk, Wv, bv, pe_W1, pe_b1, pe_W2, pe_b2,
           gru_Wih, gru_Whh, gru_bih, gru_bhh, mlp_W1, mlp_b1, mlp_W2, mlp_b2,
           ln_in_g, ln_in_b, ln_s_g, ln_s_b, ln_m_g, ln_m_b):
    inp2 = inputs.reshape(_BN, _D)
    pos3 = jnp.transpose(pos, (0, 2, 1))
    row = lambda v: v.reshape(1, -1)

    vfeat, kt = _k1(inp2, Wv, row(bv), Wk, row(bk), row(ln_in_g), row(ln_in_b))
    ksum3 = kt.reshape(_B, 1, _N)
    fars, spos = _k2(pos3)
    slots0 = _k3(fars, inp2, row(ln_in_g), row(ln_in_b))
    sel, ksn, pes = _k4(pos3, spos, ksum3, pe_W1.T, row(pe_b1), pe_W2,
                        row(pe_b2))
    slots = _k5(slots0, sel, ksn, pes, vfeat, Wq, row(bq),
                gru_Wih, gru_Whh, row(gru_bih), row(gru_bhh),
                mlp_W1, row(mlp_b1), mlp_W2, row(mlp_b2),
                row(ln_s_g), row(ln_s_b), row(ln_m_g), row(ln_m_b))
    return slots.reshape(_B, _S, _D), spos.reshape(_B, _S, 3)


# speedup vs baseline: 2.7204x; 1.0228x over previous
import jax
import jax.numpy as jnp
from jax.experimental import pallas as pl

_B, _N, _D = 4, 4096, 256
_S, _K, _ITERS, _H = 64, 16, 3, 128
_BN = _B * _N
_BS = _B * _S
_CH = 2048
_NC = _BN // _CH
_PREC = jax.lax.Precision.HIGHEST


def _ln(x, g, b, eps=1e-5):
    m = jnp.mean(x, axis=-1, keepdims=True)
    xc = x - m
    v = jnp.mean(xc * xc, axis=-1, keepdims=True)
    return xc / jnp.sqrt(v + eps) * g + b


def _dot_nt(a, b):
    return jax.lax.dot_general(a, b, (((1,), (1,)), ((), ())), precision=_PREC)


def _dot_nn(a, b):
    return jax.lax.dot_general(a, b, (((1,), (0,)), ((), ())), precision=_PREC)


def _proj_body(x_ref, Wv_ref, bv_ref, Wk_ref, bk_ref, g_ref, b_ref,
               v_ref, k_ref):
    xn = _ln(x_ref[...], g_ref[...], b_ref[...])
    v_ref[...] = _dot_nt(xn, Wv_ref[...]) + bv_ref[...]
    wkc = jnp.sum(Wk_ref[...], axis=0, keepdims=True)
    k_ref[0] = _dot_nt(wkc, xn) + jnp.sum(bk_ref[...])


def _k1(inp2, Wv, bv, Wk, bk, g, b):
    return pl.pallas_call(
        _proj_body,
        grid=(_NC,),
        in_specs=[
            pl.BlockSpec((_CH, _D), lambda c: (c, 0)),
            pl.BlockSpec((_D, _D), lambda c: (0, 0)),
            pl.BlockSpec((1, _D), lambda c: (0, 0)),
            pl.BlockSpec((_D, _D), lambda c: (0, 0)),
            pl.BlockSpec((1, _D), lambda c: (0, 0)),
            pl.BlockSpec((1, _D), lambda c: (0, 0)),
            pl.BlockSpec((1, _D), lambda c: (0, 0)),
        ],
        out_specs=[
            pl.BlockSpec((_CH, _D), lambda c: (c, 0)),
            pl.BlockSpec((1, 1, _CH), lambda c: (c, 0, 0)),
        ],
        out_shape=[
            jax.ShapeDtypeStruct((_BN, _D), jnp.float32),
            jax.ShapeDtypeStruct((_NC, 1, _CH), jnp.float32),
        ],
    )(inp2, Wv, bv, Wk, bk, g, b)


def _fps_topk_body(pos3_ref, x_ref, ksum_ref, pe_W1T_ref, pe_b1_ref,
                   pe_W2_ref, pe_b2_ref, g_ref, b_ref,
                   slots0_ref, spos_ref, sel_ref, ksn_ref, pes_ref,
                   fars_ref):
    colN = jax.lax.broadcasted_iota(jnp.int32, (_B, _N), 1)
    colS = jax.lax.broadcasted_iota(jnp.int32, (_S, _N), 1)
    colF = jax.lax.broadcasted_iota(jnp.int32, (_S, 128), 1)
    rowF = jax.lax.broadcasted_iota(jnp.int32, (_S, 128), 0)
    px = jnp.concatenate([pos3_ref[b, 0:1, :] for b in range(_B)], axis=0)
    py = jnp.concatenate([pos3_ref[b, 1:2, :] for b in range(_B)], axis=0)
    pz = jnp.concatenate([pos3_ref[b, 2:3, :] for b in range(_B)], axis=0)

    fars_ref[...] = jnp.zeros((_S, 128), jnp.float32)

    def fps_body(t, carry):
        dist, far = carry
        rowt = rowF == t
        upd = jnp.zeros((_S, 128), jnp.float32)
        for b in range(_B):
            fb = far[b:b + 1, 0:1].astype(jnp.float32)
            upd = upd + jnp.where(rowt & (colF == b), fb, 0.0)
        fars_ref[...] += upd
        selN = colN == far
        cx = jnp.sum(jnp.where(selN, px, 0.0), axis=1, keepdims=True)
        cy = jnp.sum(jnp.where(selN, py, 0.0), axis=1, keepdims=True)
        cz = jnp.sum(jnp.where(selN, pz, 0.0), axis=1, keepdims=True)
        d = (px - cx) ** 2 + (py - cy) ** 2 + (pz - cz) ** 2
        dist = jnp.minimum(dist, d)
        m = jnp.max(dist, axis=1, keepdims=True)
        far = jnp.min(jnp.where(dist == m, colN, _N), axis=1, keepdims=True)
        return dist, far.astype(jnp.int32)

    dist0 = jnp.full((_B, _N), 1e10, jnp.float32)
    far0 = jnp.zeros((_B, 1), jnp.int32)
    jax.lax.fori_loop(0, _S, fps_body, (dist0, far0))

    w1x = pe_W1T_ref[0:1, :]
    w1y = pe_W1T_ref[1:2, :]
    w1z = pe_W1T_ref[2:3, :]
    pe_b1 = pe_b1_ref[...]
    pe_c = jnp.sum(pe_W2_ref[...], axis=0, keepdims=True)
    pe_const = jnp.sum(pe_b2_ref[...])

    for b in range(_B):
        sl = slice(b * _S, (b + 1) * _S)
        idx = fars_ref[:, b:b + 1].astype(jnp.int32)
        ohb = (colS == idx).astype(jnp.float32)
        spos_b = _dot_nt(ohb, pos3_ref[b])
        spos_ref[sl, :] = spos_b
        raw = _dot_nn(ohb, x_ref[b * _N:(b + 1) * _N, :])
        slots0_ref[sl, :] = _ln(raw, g_ref[...], b_ref[...])

        pxb = pos3_ref[b, 0:1, :]
        pyb = pos3_ref[b, 1:2, :]
        pzb = pos3_ref[b, 2:3, :]
        spx = spos_b[:, 0:1]
        spy = spos_b[:, 1:2]
        spz = spos_b[:, 2:3]
        work = (spx - pxb) ** 2 + (spy - pyb) ** 2 + (spz - pzb) ** 2
        tab = jnp.concatenate([pxb, pyb, pzb, ksum_ref[b]], axis=0)

        for r in range(_K):
            mn = jnp.min(work, axis=1, keepdims=True)
            sel = jnp.min(jnp.where(work == mn, colS, _N), axis=1, keepdims=True)
            ohr = (colS == sel).astype(jnp.float32)
            gf = _dot_nt(ohr, tab)
            work = jnp.where(colS == sel, 1e30, work)
            sel_ref[sl, r:r + 1] = sel
            ksn_ref[sl, r:r + 1] = gf[:, 3:4]
            hr = jax.nn.relu((spx - gf[:, 0:1]) * w1x + (spy - gf[:, 1:2]) * w1y
                             + (spz - gf[:, 2:3]) * w1z + pe_b1)
            pes_ref[sl, r:r + 1] = (jnp.sum(hr * pe_c, axis=1, keepdims=True)
                                    + pe_const)


def _k2(pos3, inp2, ksum3, pe_W1T, pe_b1, pe_W2, pe_b2, g, b):
    from jax.experimental.pallas import tpu as pltpu
    return pl.pallas_call(
        _fps_topk_body,
        out_shape=[
            jax.ShapeDtypeStruct((_BS, _D), jnp.float32),
            jax.ShapeDtypeStruct((_BS, 3), jnp.float32),
            jax.ShapeDtypeStruct((_BS, _K), jnp.int32),
            jax.ShapeDtypeStruct((_BS, _K), jnp.float32),
            jax.ShapeDtypeStruct((_BS, _K), jnp.float32),
        ],
        scratch_shapes=[pltpu.VMEM((_S, 128), jnp.float32)],
    )(pos3, inp2, ksum3.reshape(_B, 1, _N), pe_W1T, pe_b1, pe_W2, pe_b2, g, b)


def _iter_body(slots0_ref, sel_ref, ksn_ref, pes_ref, vfeat_ref,
               Wq_ref, bq_ref,
               gru_Wih_ref, gru_Whh_ref, gru_bih_ref, gru_bhh_ref,
               mlp_W1_ref, mlp_b1_ref, mlp_W2_ref, mlp_b2_ref,
               ln_s_g_ref, ln_s_b_ref, ln_m_g_ref, ln_m_b_ref,
               out_ref):
    slots = slots0_ref[...]
    ksn = ksn_ref[...]
    pesum = pes_ref[...]
    selb = sel_ref[...]
    colS = jax.lax.broadcasted_iota(jnp.int32, (_S, _N), 1)

    for _ in range(_ITERS):
        slots_prev = slots
        sn = _ln(slots, ln_s_g_ref[...], ln_s_b_ref[...])
        q = _dot_nt(sn, Wq_ref[...]) + bq_ref[...]
        qsum = jnp.sum(q, axis=1, keepdims=True)

        scores = qsum - ksn + pesum
        smax = jnp.max(scores, axis=1, keepdims=True)
        e = jnp.exp(scores - smax)
        a = e / jnp.sum(e, axis=1, keepdims=True)
        a = a / (jnp.sum(a, axis=0, keepdims=True) + 1e-6)

        amat = jnp.zeros((_S, _N), jnp.float32)
        for r in range(_K):
            amat = amat + jnp.where(colS == selb[:, r:r + 1],
                                    a[:, r:r + 1], 0.0)
        upd = _dot_nn(amat, vfeat_ref[...])

        gi = _dot_nt(upd, gru_Wih_ref[...]) + gru_bih_ref[...]
        gh = _dot_nt(slots_prev, gru_Whh_ref[...]) + gru_bhh_ref[...]
        i_r = gi[:, :_D]
        i_z = gi[:, _D:2 * _D]
        i_n = gi[:, 2 * _D:]
        h_r = gh[:, :_D]
        h_z = gh[:, _D:2 * _D]
        h_n = gh[:, 2 * _D:]
        r_g = jax.nn.sigmoid(i_r + h_r)
        z_g = jax.nn.sigmoid(i_z + h_z)
        n_g = jnp.tanh(i_n + r_g * h_n)
        slots = (1.0 - z_g) * n_g + z_g * slots_prev

        mid = jax.nn.relu(
            _dot_nt(_ln(slots, ln_m_g_ref[...], ln_m_b_ref[...]), mlp_W1_ref[...])
            + mlp_b1_ref[...])
        slots = slots + _dot_nt(mid, mlp_W2_ref[...]) + mlp_b2_ref[...]

    out_ref[...] = slots


def _k5(slots0, sel, ksn, pes, vfeat, Wq, bq, gru_Wih, gru_Whh, gru_bih,
        gru_bhh, mlp_W1, mlp_b1, mlp_W2, mlp_b2, ln_s_g, ln_s_b,
        ln_m_g, ln_m_b):
    w = lambda shape: pl.BlockSpec(shape, lambda i: (0,) * len(shape))
    return pl.pallas_call(
        _iter_body,
        grid=(_B,),
        in_specs=[
            pl.BlockSpec((_S, _D), lambda i: (i, 0)),
            pl.BlockSpec((_S, _K), lambda i: (i, 0)),
            pl.BlockSpec((_S, _K), lambda i: (i, 0)),
            pl.BlockSpec((_S, _K), lambda i: (i, 0)),
            pl.BlockSpec((_N, _D), lambda i: (i, 0)),
            w((_D, _D)), w((1, _D)),
            w((3 * _D, _D)), w((3 * _D, _D)), w((1, 3 * _D)), w((1, 3 * _D)),
            w((_H, _D)), w((1, _H)), w((_D, _H)), w((1, _D)),
            w((1, _D)), w((1, _D)), w((1, _D)), w((1, _D)),
        ],
        out_specs=pl.BlockSpec((_S, _D), lambda i: (i, 0)),
        out_shape=jax.ShapeDtypeStruct((_BS, _D), jnp.float32),
    )(slots0, sel, ksn, pes, vfeat, Wq, bq, gru_Wih, gru_Whh, gru_bih,
      gru_bhh, mlp_W1, mlp_b1, mlp_W2, mlp_b2, ln_s_g, ln_s_b,
      ln_m_g, ln_m_b)


def kernel(inputs, pos, Wq, bq, Wk, bk, Wv, bv, pe_W1, pe_b1, pe_W2, pe_b2,
           gru_Wih, gru_Whh, gru_bih, gru_bhh, mlp_W1, mlp_b1, mlp_W2, mlp_b2,
           ln_in_g, ln_in_b, ln_s_g, ln_s_b, ln_m_g, ln_m_b):
    inp2 = inputs.reshape(_BN, _D)
    pos3 = jnp.transpose(pos, (0, 2, 1))
    row = lambda v: v.reshape(1, -1)

    vfeat, kt = _k1(inp2, Wv, row(bv), Wk, row(bk), row(ln_in_g), row(ln_in_b))
    slots0, spos, sel, ksn, pes = _k2(
        pos3, inp2, kt, pe_W1.T, row(pe_b1), pe_W2, row(pe_b2),
        row(ln_in_g), row(ln_in_b))
    slots = _k5(slots0, sel, ksn, pes, vfeat, Wq, row(bq),
                gru_Wih, gru_Whh, row(gru_bih), row(gru_bhh),
                mlp_W1, row(mlp_b1), mlp_W2, row(mlp_b2),
                row(ln_s_g), row(ln_s_b), row(ln_m_g), row(ln_m_b))
    return slots.reshape(_B, _S, _D), spos.reshape(_B, _S, 3)
